# baseline (speedup 1.0000x reference)
import functools
import math

import jax
import jax.numpy as jnp
from jax import lax
from jax.experimental import pallas as pl
from jax.experimental.pallas import tpu as pltpu
from jax.experimental.pallas import tpu_sc as plsc

D = 64
B = 16384
SCALE = 1.0 / math.sqrt(4.0)

NC = 2
NS = 16
NW = NC * NS
BPW = B // NW
CH = 128
NCHUNK = BPW // CH

TBLK = 1024


def _repack_body(et_ref, eye_ref, o_ref):
    o_ref[...] = lax.dot_general(et_ref[...], eye_ref[...],
                                 (((0,), (0,)), ((), ())),
                                 preferred_element_type=jnp.float32)


def _repack(table_t, eye):
    rows = table_t.shape[1]
    grid = (rows + TBLK - 1) // TBLK
    return pl.pallas_call(
        _repack_body,
        grid=(grid,),
        in_specs=[
            pl.BlockSpec((D, TBLK), lambda q: (0, q)),
            pl.BlockSpec((D, 128), lambda q: (0, 0)),
        ],
        out_specs=pl.BlockSpec((TBLK, 128), lambda q: (q, 0)),
        out_shape=jax.ShapeDtypeStruct((grid * TBLK, 128), jnp.float32),
    )(table_t, eye)


def _adj_body(dt_ref, w_ref, o_ref):
    acc = lax.dot_general(dt_ref[...], w_ref[...], (((0,), (0,)), ((), ())),
                          preferred_element_type=jnp.float32)
    o_ref[...] = acc * SCALE


def _adjustment(dense_t, W_dense):
    return pl.pallas_call(
        _adj_body,
        out_shape=jax.ShapeDtypeStruct((B, D), jnp.float32),
    )(dense_t, W_dense)


def _sc_body(uid_hbm, iid_hbm, cid_hbm, adj_hbm, eu_hbm, ei_hbm, ec_hbm,
             out_hbm, idxu_v, idxi_v, idxc_v, ru_v, ri_v, rc_v,
             adj_v, sem, adj_sem):
    wid = lax.axis_index("s") * NC + lax.axis_index("c")
    base = wid * BPW

    def chunk(ci, carry):
        off = base + ci * CH
        pltpu.sync_copy(uid_hbm.at[pl.ds(off, CH)], idxu_v)
        pltpu.sync_copy(iid_hbm.at[pl.ds(off, CH)], idxi_v)
        pltpu.sync_copy(cid_hbm.at[pl.ds(off, CH)], idxc_v)
        ca = pltpu.async_copy(adj_hbm.at[pl.ds(off, CH)], adj_v, adj_sem)
        cu = pltpu.async_copy(eu_hbm.at[idxu_v], ru_v, sem)
        cit = pltpu.async_copy(ei_hbm.at[idxi_v], ri_v, sem)
        cc = pltpu.async_copy(ec_hbm.at[idxc_v], rc_v, sem)
        cu.wait()
        cit.wait()
        cc.wait()
        ca.wait()

        def combine(g, rcarry):
            vu = idxu_v[pl.ds(g * 16, 16)]
            vi = idxi_v[pl.ds(g * 16, 16)]
            vc = idxc_v[pl.ds(g * 16, 16)]
            for l in range(16):
                r = g * 16 + l
                su = jnp.where(vu[l] == 0, 0.0, SCALE)
                si = jnp.where(vi[l] == 0, 0.0, SCALE)
                sc = jnp.where(vc[l] == 0, 0.0, SCALE)
                for c4 in range(D // 16):
                    s = pl.ds(c4 * 16, 16)
                    adj_v[r, s] = (ru_v[r, s] * su + ri_v[r, s] * si
                                   + rc_v[r, s] * sc + adj_v[r, s])
            return rcarry

        lax.fori_loop(0, CH // 16, combine, 0)
        pltpu.sync_copy(adj_v, out_hbm.at[pl.ds(off, CH)])
        return carry

    lax.fori_loop(0, NCHUNK, chunk, 0)


_sc_call = functools.partial(
    pl.kernel,
    out_type=jax.ShapeDtypeStruct((B, D), jnp.float32),
    mesh=plsc.VectorSubcoreMesh(core_axis_name="c", subcore_axis_name="s"),
    scratch_types=[
        pltpu.VMEM((CH,), jnp.int32),
        pltpu.VMEM((CH,), jnp.int32),
        pltpu.VMEM((CH,), jnp.int32),
        pltpu.VMEM((CH, 128), jnp.float32),
        pltpu.VMEM((CH, 128), jnp.float32),
        pltpu.VMEM((CH, 128), jnp.float32),
        pltpu.VMEM((CH, D), jnp.float32),
        pltpu.SemaphoreType.DMA,
        pltpu.SemaphoreType.DMA,
    ],
)(_sc_body)


def kernel(user_id, item_id, category, dense_0, E_user, E_item, E_cat,
           W_dense):
    u = user_id.astype(jnp.int32)
    i = item_id.astype(jnp.int32)
    c = category.astype(jnp.int32)
    eye = jnp.eye(D, 128, dtype=jnp.float32)
    eu2 = _repack(E_user.T, eye)
    ei2 = _repack(E_item.T, eye)
    ec2 = _repack(E_cat.T, eye)
    adj = _adjustment(dense_0.T, W_dense)
    return _sc_call(u, i, c, adj, eu2, ei2, ec2)


# speedup vs baseline: 3.0276x; 3.0276x over previous
import functools
import math

import jax
import jax.numpy as jnp
from jax import lax
from jax.experimental import pallas as pl
from jax.experimental.pallas import tpu as pltpu
from jax.experimental.pallas import tpu_sc as plsc

D = 64
B = 16384
SCALE = 1.0 / math.sqrt(4.0)

NC = 2
NS = 16
NW = NC * NS
BPW = B // NW
CH = 128
NCHUNK = BPW // CH


def _adj_body(dt_ref, w_ref, o_ref):
    acc = lax.dot_general(dt_ref[...], w_ref[...], (((0,), (0,)), ((), ())),
                          preferred_element_type=jnp.float32)
    o_ref[...] = acc * SCALE


def _adjustment(dense_t, W_dense):
    return pl.pallas_call(
        _adj_body,
        out_shape=jax.ShapeDtypeStruct((B, D), jnp.float32),
    )(dense_t, W_dense)


def _sc_body(uid_hbm, iid_hbm, cid_hbm, adj_hbm, eu_hbm, ei_hbm, ec_hbm,
             out_hbm, idxu_v, idxi_v, idxc_v, ru_v, ri_v, rc_v,
             adj_v, sem, adj_sem):
    wid = lax.axis_index("s") * NC + lax.axis_index("c")
    base = wid * BPW

    def chunk(ci, carry):
        off = base + ci * CH
        pltpu.sync_copy(uid_hbm.at[pl.ds(off, CH)], idxu_v)
        pltpu.sync_copy(iid_hbm.at[pl.ds(off, CH)], idxi_v)
        pltpu.sync_copy(cid_hbm.at[pl.ds(off, CH)], idxc_v)
        ca = pltpu.async_copy(adj_hbm.at[pl.ds(off, CH)], adj_v, adj_sem)

        def issue(g, rcarry):
            vu = idxu_v[pl.ds(g * 16, 16)]
            vi = idxi_v[pl.ds(g * 16, 16)]
            vc = idxc_v[pl.ds(g * 16, 16)]
            for l in range(16):
                r = g * 16 + l
                pltpu.async_copy(eu_hbm.at[pl.ds(vu[l], 1)],
                                 ru_v.at[pl.ds(r, 1)], sem)
                pltpu.async_copy(ei_hbm.at[pl.ds(vi[l], 1)],
                                 ri_v.at[pl.ds(r, 1)], sem)
                pltpu.async_copy(ec_hbm.at[pl.ds(vc[l], 1)],
                                 rc_v.at[pl.ds(r, 1)], sem)
            return rcarry

        lax.fori_loop(0, CH // 16, issue, 0)
        pltpu.make_async_copy(eu_hbm.at[pl.ds(0, CH)], ru_v, sem).wait()
        pltpu.make_async_copy(ei_hbm.at[pl.ds(0, CH)], ri_v, sem).wait()
        pltpu.make_async_copy(ec_hbm.at[pl.ds(0, CH)], rc_v, sem).wait()
        ca.wait()

        def row(r, rcarry):
            for c4 in range(D // 16):
                s = pl.ds(c4 * 16, 16)
                adj_v[r, s] = (ru_v[r, s] + ri_v[r, s] + rc_v[r, s]) * SCALE \
                    + adj_v[r, s]
            return rcarry

        lax.fori_loop(0, CH, row, 0)
        pltpu.sync_copy(adj_v, out_hbm.at[pl.ds(off, CH)])
        return carry

    lax.fori_loop(0, NCHUNK, chunk, 0)


_sc_call = functools.partial(
    pl.kernel,
    out_type=jax.ShapeDtypeStruct((B, D), jnp.float32),
    mesh=plsc.VectorSubcoreMesh(core_axis_name="c", subcore_axis_name="s"),
    scratch_types=[
        pltpu.VMEM((CH,), jnp.int32),
        pltpu.VMEM((CH,), jnp.int32),
        pltpu.VMEM((CH,), jnp.int32),
        pltpu.VMEM((CH, D), jnp.float32),
        pltpu.VMEM((CH, D), jnp.float32),
        pltpu.VMEM((CH, D), jnp.float32),
        pltpu.VMEM((CH, D), jnp.float32),
        pltpu.SemaphoreType.DMA,
        pltpu.SemaphoreType.DMA,
    ],
)(_sc_body)


def kernel(user_id, item_id, category, dense_0, E_user, E_item, E_cat,
           W_dense):
    u = user_id.astype(jnp.int32)
    i = item_id.astype(jnp.int32)
    c = category.astype(jnp.int32)
    Eu = E_user.at[0].set(0.0)
    Ei = E_item.at[0].set(0.0)
    Ec = E_cat.at[0].set(0.0)
    adj = _adjustment(dense_0.T, W_dense)
    return _sc_call(u, i, c, adj, Eu, Ei, Ec)
